# baseline (speedup 1.0000x reference)
import functools

import jax
import jax.numpy as jnp
from jax import lax
from jax.experimental import pallas as pl
from jax.experimental.pallas import tpu as pltpu
from jax.experimental.pallas import tpu_sc as plsc

K = 3
M = 4
N = 2048
D = 128


def _sc_adj_body(dst_hbm, src_hbm, a_hbm, flat_v, src_v, slab, half_id=0):
    wid = lax.axis_index("s")

    e_total = dst_hbm.shape[0]
    rows_per_tile = N // 32
    hw = N // 2
    slab_words = rows_per_tile * hw

    pltpu.sync_copy(dst_hbm, flat_v)
    pltpu.sync_copy(src_hbm, src_v)

    zeros16 = jnp.zeros((16,), jnp.int32)
    i32 = jnp.int32

    unroll = 8

    def _pack(i, _):
        for u in range(unroll):
            off = i * i32(16 * unroll) + i32(16 * u)
            s = src_v[pl.ds(off, 16)]
            flat_v[pl.ds(off, 16)] = (
                flat_v[pl.ds(off, 16)] * i32(N)
                + (s & i32(hw - 1)) * i32(2)
                + (s >> i32(10))
            )
        return _

    lax.fori_loop(i32(0), i32(e_total // (16 * unroll)), _pack, None)

    def _zero(i, _):
        for u in range(unroll):
            slab[pl.ds(i * i32(16 * unroll) + i32(16 * u), 16)] = zeros16
        return _

    lax.fori_loop(i32(0), i32(slab_words // (16 * unroll)), _zero, None)

    base2 = (wid + i32(16 * half_id)) * i32(2 * slab_words)
    for half, val in ((0, 1), (1, 1 << 16)):
        val16 = jnp.full((16,), val, jnp.int32)

        def _scan(e, _):
            for u in range(unroll):
                off = e * i32(16 * unroll) + i32(16 * u)
                local2 = flat_v[pl.ds(off, 16)] - base2
                inslab = plsc.bitcast(local2, jnp.uint32) < jnp.uint32(2 * slab_words)
                mask = inslab & ((local2 & i32(1)) == i32(half))
                idx = jnp.where(mask, local2 >> i32(1), i32(0))
                plsc.addupdate_scatter(slab, [idx], val16, mask=mask)
            return _

        lax.fori_loop(i32(0), i32(e_total // (16 * unroll)), _scan, None)

    pltpu.sync_copy(slab, a_hbm.at[pl.ds(wid * i32(slab_words), slab_words)])


def _sc_adj_half(dst_i32, src_i32, half_id):
    mesh = plsc.VectorSubcoreMesh(
        core_axis_name="c", subcore_axis_name="s", num_cores=1
    )
    e_total = dst_i32.shape[0]
    f = functools.partial(
        pl.kernel,
        mesh=mesh,
        out_type=jax.ShapeDtypeStruct((N * (N // 2) // 2,), jnp.int32),
        scratch_types=[
            pltpu.VMEM((e_total,), jnp.int32),
            pltpu.VMEM((e_total,), jnp.int32),
            pltpu.VMEM(((N // 32) * (N // 2),), jnp.int32),
        ],
        compiler_params=pltpu.CompilerParams(needs_layout_passes=False),
    )(functools.partial(_sc_adj_body, half_id=half_id))
    return f(dst_i32, src_i32)


def _sc_build_adj(dst_i32, src_i32):
    mesh = plsc.VectorSubcoreMesh(core_axis_name="c", subcore_axis_name="s")
    e_total = dst_i32.shape[0]
    f = functools.partial(
        pl.kernel,
        mesh=mesh,
        out_type=jax.ShapeDtypeStruct((N * (N // 2),), jnp.int32),
        scratch_types=[
            pltpu.VMEM((e_total,), jnp.int32),
            pltpu.VMEM((e_total,), jnp.int32),
            pltpu.VMEM(((N // 32) * (N // 2),), jnp.int32),
        ],
        compiler_params=pltpu.CompilerParams(needs_layout_passes=False),
    )(_sc_adj_body)
    return f(dst_i32, src_i32)


def _prep_body(x_ref, mo_ref):
    a = jnp.abs(x_ref[...])
    for b in range(a.shape[0]):
        p = a[b]
        for m in range(M):
            c0 = (b * M + m) * D
            mo_ref[:, c0:c0 + D] = p.astype(jnp.bfloat16)
            if m + 1 < M:
                p = p * a[b]


def _tc_prep(x):
    b = x.shape[0]
    return pl.pallas_call(
        _prep_body,
        out_shape=jax.ShapeDtypeStruct((N, b * M * D), jnp.bfloat16),
    )(x)


def _main_body(at_ref, ab_ref, mo_ref, out_ref, abf):
    @pl.when(pl.program_id(0) == 0)
    def _cast():
        hw = N // 2
        for rref, r0 in ((at_ref, 0), (ab_ref, N // 2)):
            aw = rref[...]
            abf[r0:r0 + N // 2, :hw] = ((aw & 0xFFFF) != 0).astype(jnp.bfloat16)
            abf[r0:r0 + N // 2, hw:] = ((aw >> 16) != 0).astype(jnp.bfloat16)

    rr = out_ref.shape[1]
    nb = out_ref.shape[0]
    row0 = pl.program_id(0) * rr
    a = abf[...]
    mo = mo_ref[...]
    hr = rr // 2
    halves = []
    for h in range(2):
        r0h = row0 + h * hr
        rows = lax.broadcasted_iota(jnp.int32, (hr, N), 0) + r0h
        cols = lax.broadcasted_iota(jnp.int32, (hr, N), 1)
        halves.append((rows == cols) | (abf[pl.ds(r0h, hr), :] > 0))
    for k in range(K):
        for h in range(2):
            r_bf = halves[h].astype(jnp.bfloat16)
            s = jnp.dot(r_bf, mo, preferred_element_type=jnp.float32)
            for b in range(nb):
                out_ref[b, h * hr:(h + 1) * hr, k] = (
                    s[:, b * M * D:(b + 1) * M * D].reshape(hr, M, D)
                )
            if k + 1 < K:
                ra = jnp.dot(r_bf, a, preferred_element_type=jnp.float32)
                halves[h] = halves[h] | (ra > 0.0)


def _tc_main(a_top, a_bot, mo, nb):
    c = mo.shape[1]
    rr = 512
    return pl.pallas_call(
        _main_body,
        grid=(N // rr,),
        in_specs=[
            pl.BlockSpec((N // 2, N // 2), lambda i: (i * 0, i * 0)),
            pl.BlockSpec((N // 2, N // 2), lambda i: (i * 0, i * 0)),
            pl.BlockSpec((N, c), lambda i: (i * 0, i * 0)),
        ],
        out_specs=pl.BlockSpec(
            (nb, rr, K, M, D), lambda i: (i * 0, i, i * 0, i * 0, i * 0)
        ),
        out_shape=jax.ShapeDtypeStruct((nb, N, K, M, D), jnp.float32),
        scratch_shapes=[pltpu.VMEM((N, N), jnp.bfloat16)],
        compiler_params=pltpu.CompilerParams(
            vmem_limit_bytes=100 * 1024 * 1024,
        ),
    )(a_top, a_bot, mo)


def kernel(x, edge_index):
    b = x.shape[0]
    e32 = edge_index.astype(jnp.int32)
    top = _sc_adj_half(e32[1], e32[0], 0).reshape(N // 2, N // 2)
    bot = _sc_adj_half(e32[1], e32[0], 1).reshape(N // 2, N // 2)
    mo = _tc_prep(x)
    return _tc_main(top, bot, mo, b)


# speedup vs baseline: 1.2401x; 1.2401x over previous
import functools

import jax
import jax.numpy as jnp
from jax import lax
from jax.experimental import pallas as pl
from jax.experimental.pallas import tpu as pltpu
from jax.experimental.pallas import tpu_sc as plsc

K = 3
M = 4
N = 2048
D = 128


def _sc_adj_body(dst_hbm, src_hbm, a_hbm, flat_v, src_v, slab):
    num_cores = 2
    wid = lax.axis_index("s") * num_cores + lax.axis_index("c")

    e_total = dst_hbm.shape[0]
    rows_per_tile = N // 32
    hw = N // 2
    slab_words = rows_per_tile * hw

    pltpu.sync_copy(dst_hbm, flat_v)
    pltpu.sync_copy(src_hbm, src_v)

    zeros16 = jnp.zeros((16,), jnp.int32)
    i32 = jnp.int32

    unroll = 8

    def _pack(i, _):
        for u in range(unroll):
            off = i * i32(16 * unroll) + i32(16 * u)
            s = src_v[pl.ds(off, 16)]
            flat_v[pl.ds(off, 16)] = (
                flat_v[pl.ds(off, 16)] * i32(N)
                + (s & i32(hw - 1)) * i32(2)
                + (s >> i32(10))
            )
        return _

    lax.fori_loop(i32(0), i32(e_total // (16 * unroll)), _pack, None)

    def _zero(i, _):
        for u in range(unroll):
            slab[pl.ds(i * i32(16 * unroll) + i32(16 * u), 16)] = zeros16
        return _

    lax.fori_loop(i32(0), i32(slab_words // (16 * unroll)), _zero, None)

    base2 = wid * i32(2 * slab_words)
    one16 = jnp.full((16,), 1, jnp.int32)
    hi16 = jnp.full((16,), 1 << 16, jnp.int32)

    def _scan(e, _):
        for u in range(unroll):
            off = e * i32(16 * unroll) + i32(16 * u)
            local2 = flat_v[pl.ds(off, 16)] - base2
            inslab = plsc.bitcast(local2, jnp.uint32) < jnp.uint32(2 * slab_words)
            odd = (local2 & i32(1)) == i32(1)
            idx = jnp.where(inslab, local2 >> i32(1), i32(0))
            mlo = inslab & jnp.logical_not(odd)
            mhi = inslab & odd
            plsc.addupdate_scatter(slab, [idx], one16, mask=mlo)
            plsc.addupdate_scatter(slab, [idx], hi16, mask=mhi)
        return _

    lax.fori_loop(i32(0), i32(e_total // (16 * unroll)), _scan, None)

    pltpu.sync_copy(slab, a_hbm.at[pl.ds(wid * i32(slab_words), slab_words)])


def _sc_build_adj(dst_i32, src_i32):
    mesh = plsc.VectorSubcoreMesh(core_axis_name="c", subcore_axis_name="s")
    e_total = dst_i32.shape[0]
    f = functools.partial(
        pl.kernel,
        mesh=mesh,
        out_type=jax.ShapeDtypeStruct((N * (N // 2),), jnp.int32),
        scratch_types=[
            pltpu.VMEM((e_total,), jnp.int32),
            pltpu.VMEM((e_total,), jnp.int32),
            pltpu.VMEM(((N // 32) * (N // 2),), jnp.int32),
        ],
        compiler_params=pltpu.CompilerParams(needs_layout_passes=False),
    )(_sc_adj_body)
    return f(dst_i32, src_i32)


def _prep_body(x_ref, mo_ref):
    a = jnp.abs(x_ref[...])
    for b in range(a.shape[0]):
        p = a[b]
        for m in range(M):
            c0 = (b * M + m) * D
            mo_ref[:, c0:c0 + D] = p.astype(jnp.bfloat16)
            if m + 1 < M:
                p = p * a[b]


def _tc_prep(x):
    b = x.shape[0]
    return pl.pallas_call(
        _prep_body,
        out_shape=jax.ShapeDtypeStruct((N, b * M * D), jnp.bfloat16),
    )(x)


def _main_body(a_ref, mo_ref, out_ref, abf):
    @pl.when(pl.program_id(0) == 0)
    def _cast():
        aw = a_ref[...]
        hw = N // 2
        abf[:, :hw] = ((aw & 0xFFFF) != 0).astype(jnp.bfloat16)
        abf[:, hw:] = ((aw >> 16) != 0).astype(jnp.bfloat16)

    rr = out_ref.shape[1]
    nb = out_ref.shape[0]
    row0 = pl.program_id(0) * rr
    a = abf[...]
    mo = mo_ref[...]
    hr = rr // 2
    halves = []
    for h in range(2):
        r0h = row0 + h * hr
        rows = lax.broadcasted_iota(jnp.int32, (hr, N), 0) + r0h
        cols = lax.broadcasted_iota(jnp.int32, (hr, N), 1)
        halves.append((rows == cols) | (abf[pl.ds(r0h, hr), :] > 0))
    for k in range(K):
        for h in range(2):
            r_bf = halves[h].astype(jnp.bfloat16)
            s = jnp.dot(r_bf, mo, preferred_element_type=jnp.float32)
            for b in range(nb):
                out_ref[b, h * hr:(h + 1) * hr, k] = (
                    s[:, b * M * D:(b + 1) * M * D].reshape(hr, M, D)
                )
            if k + 1 < K:
                ra = jnp.dot(r_bf, a, preferred_element_type=jnp.float32)
                halves[h] = halves[h] | (ra > 0.0)


def _tc_main(a_i32, mo, nb):
    c = mo.shape[1]
    rr = 512
    return pl.pallas_call(
        _main_body,
        grid=(N // rr,),
        in_specs=[
            pl.BlockSpec((N, N // 2), lambda i: (i * 0, i * 0)),
            pl.BlockSpec((N, c), lambda i: (i * 0, i * 0)),
        ],
        out_specs=pl.BlockSpec(
            (nb, rr, K, M, D), lambda i: (i * 0, i, i * 0, i * 0, i * 0)
        ),
        out_shape=jax.ShapeDtypeStruct((nb, N, K, M, D), jnp.float32),
        scratch_shapes=[pltpu.VMEM((N, N), jnp.bfloat16)],
        compiler_params=pltpu.CompilerParams(
            vmem_limit_bytes=100 * 1024 * 1024,
        ),
    )(a_i32, mo)


def kernel(x, edge_index):
    b = x.shape[0]
    e32 = edge_index.astype(jnp.int32)
    a_i32 = _sc_build_adj(e32[1], e32[0]).reshape(N, N // 2)
    mo = _tc_prep(x)
    return _tc_main(a_i32, mo, b)


# speedup vs baseline: 1.2496x; 1.0077x over previous
import functools

import jax
import jax.numpy as jnp
from jax import lax
from jax.experimental import pallas as pl
from jax.experimental.pallas import tpu as pltpu
from jax.experimental.pallas import tpu_sc as plsc

K = 3
M = 4
N = 2048
D = 128


def _sc_adj_body(dst_hbm, src_hbm, a_hbm, flat_v, src_v, slab, sem):
    num_cores = 2
    wid = lax.axis_index("s") * num_cores + lax.axis_index("c")

    e_total = dst_hbm.shape[0]
    rows_per_tile = N // 32
    hw = N // 2
    slab_words = rows_per_tile * hw

    cp_d = pltpu.make_async_copy(dst_hbm, flat_v, sem)
    cp_s = pltpu.make_async_copy(src_hbm, src_v, sem)
    cp_d.start()
    cp_s.start()
    cp_d.wait()
    cp_s.wait()

    zeros16 = jnp.zeros((16,), jnp.int32)
    i32 = jnp.int32

    unroll = 16

    def _pack(i, _):
        for u in range(unroll):
            off = i * i32(16 * unroll) + i32(16 * u)
            s = src_v[pl.ds(off, 16)]
            flat_v[pl.ds(off, 16)] = (
                flat_v[pl.ds(off, 16)] * i32(N)
                + (s & i32(hw - 1)) * i32(2)
                + (s >> i32(10))
            )
        return _

    lax.fori_loop(i32(0), i32(e_total // (16 * unroll)), _pack, None)

    def _zero(i, _):
        for u in range(unroll):
            slab[pl.ds(i * i32(16 * unroll) + i32(16 * u), 16)] = zeros16
        return _

    lax.fori_loop(i32(0), i32(slab_words // (16 * unroll)), _zero, None)

    base2 = wid * i32(2 * slab_words)
    one16 = jnp.full((16,), 1, jnp.int32)
    hi16 = jnp.full((16,), 1 << 16, jnp.int32)

    def _scan(e, _):
        for u in range(unroll):
            off = e * i32(16 * unroll) + i32(16 * u)
            local2 = flat_v[pl.ds(off, 16)] - base2
            inslab = plsc.bitcast(local2, jnp.uint32) < jnp.uint32(2 * slab_words)
            odd = (local2 & i32(1)) == i32(1)
            idx = jnp.where(inslab, local2 >> i32(1), i32(0))
            mlo = inslab & jnp.logical_not(odd)
            mhi = inslab & odd
            plsc.addupdate_scatter(slab, [idx], one16, mask=mlo)
            plsc.addupdate_scatter(slab, [idx], hi16, mask=mhi)
        return _

    lax.fori_loop(i32(0), i32(e_total // (16 * unroll)), _scan, None)

    pltpu.sync_copy(slab, a_hbm.at[pl.ds(wid * i32(slab_words), slab_words)])


def _sc_build_adj(dst_i32, src_i32):
    mesh = plsc.VectorSubcoreMesh(core_axis_name="c", subcore_axis_name="s")
    e_total = dst_i32.shape[0]
    f = functools.partial(
        pl.kernel,
        mesh=mesh,
        out_type=jax.ShapeDtypeStruct((N * (N // 2),), jnp.int32),
        scratch_types=[
            pltpu.VMEM((e_total,), jnp.int32),
            pltpu.VMEM((e_total,), jnp.int32),
            pltpu.VMEM(((N // 32) * (N // 2),), jnp.int32),
            pltpu.SemaphoreType.DMA,
        ],
        compiler_params=pltpu.CompilerParams(needs_layout_passes=False),
    )(_sc_adj_body)
    return f(dst_i32, src_i32)


def _prep_body(x_ref, mo_ref):
    a = jnp.abs(x_ref[...])
    for b in range(a.shape[0]):
        p = a[b]
        for m in range(M):
            c0 = (b * M + m) * D
            mo_ref[:, c0:c0 + D] = p.astype(jnp.bfloat16)
            if m + 1 < M:
                p = p * a[b]


def _tc_prep(x):
    b = x.shape[0]
    return pl.pallas_call(
        _prep_body,
        out_shape=jax.ShapeDtypeStruct((N, b * M * D), jnp.bfloat16),
    )(x)


def _main_body(a_ref, mo_ref, out_ref, abf):
    @pl.when(pl.program_id(0) == 0)
    def _cast():
        aw = a_ref[...]
        hw = N // 2
        abf[:, :hw] = ((aw & 0xFFFF) != 0).astype(jnp.bfloat16)
        abf[:, hw:] = ((aw >> 16) != 0).astype(jnp.bfloat16)

    rr = out_ref.shape[1]
    nb = out_ref.shape[0]
    row0 = pl.program_id(0) * rr
    a = abf[...]
    mo = mo_ref[...]
    hr = rr // 2
    halves = []
    for h in range(2):
        r0h = row0 + h * hr
        rows = lax.broadcasted_iota(jnp.int32, (hr, N), 0) + r0h
        cols = lax.broadcasted_iota(jnp.int32, (hr, N), 1)
        halves.append((rows == cols) | (abf[pl.ds(r0h, hr), :] > 0))
    for k in range(K):
        for h in range(2):
            r_bf = halves[h].astype(jnp.bfloat16)
            s = jnp.dot(r_bf, mo, preferred_element_type=jnp.float32)
            for b in range(nb):
                out_ref[b, h * hr:(h + 1) * hr, k] = (
                    s[:, b * M * D:(b + 1) * M * D].reshape(hr, M, D)
                )
            if k + 1 < K:
                ra = jnp.dot(r_bf, a, preferred_element_type=jnp.float32)
                halves[h] = halves[h] | (ra > 0.0)


def _tc_main(a_i32, mo, nb):
    c = mo.shape[1]
    rr = 512
    return pl.pallas_call(
        _main_body,
        grid=(N // rr,),
        in_specs=[
            pl.BlockSpec((N, N // 2), lambda i: (i * 0, i * 0)),
            pl.BlockSpec((N, c), lambda i: (i * 0, i * 0)),
        ],
        out_specs=pl.BlockSpec(
            (nb, rr, K, M, D), lambda i: (i * 0, i, i * 0, i * 0, i * 0)
        ),
        out_shape=jax.ShapeDtypeStruct((nb, N, K, M, D), jnp.float32),
        scratch_shapes=[pltpu.VMEM((N, N), jnp.bfloat16)],
        compiler_params=pltpu.CompilerParams(
            vmem_limit_bytes=100 * 1024 * 1024,
        ),
    )(a_i32, mo)


def kernel(x, edge_index):
    b = x.shape[0]
    e32 = edge_index.astype(jnp.int32)
    a_i32 = _sc_build_adj(e32[1], e32[0]).reshape(N, N // 2)
    mo = _tc_prep(x)
    return _tc_main(a_i32, mo, b)


# speedup vs baseline: 1.2576x; 1.0064x over previous
import functools

import jax
import jax.numpy as jnp
from jax import lax
from jax.experimental import pallas as pl
from jax.experimental.pallas import tpu as pltpu
from jax.experimental.pallas import tpu_sc as plsc

K = 3
M = 4
N = 2048
D = 128


def _sc_adj_body(dst_hbm, src_hbm, a_hbm, flat_v, src_v, slab, sem):
    num_cores = 2
    wid = lax.axis_index("s") * num_cores + lax.axis_index("c")

    e_total = dst_hbm.shape[0]
    rows_per_tile = N // 32
    hw = N // 2
    slab_words = rows_per_tile * hw

    cp_d = pltpu.make_async_copy(dst_hbm, flat_v, sem)
    cp_s = pltpu.make_async_copy(src_hbm, src_v, sem)
    cp_d.start()
    cp_s.start()
    cp_d.wait()
    cp_s.wait()

    zeros16 = jnp.zeros((16,), jnp.int32)
    i32 = jnp.int32

    unroll = 16

    def _pack(i, _):
        for u in range(unroll):
            off = i * i32(16 * unroll) + i32(16 * u)
            s = src_v[pl.ds(off, 16)]
            flat_v[pl.ds(off, 16)] = (
                flat_v[pl.ds(off, 16)] * i32(N)
                + (s & i32(hw - 1)) * i32(2)
                + (s >> i32(10))
            )
        return _

    lax.fori_loop(i32(0), i32(e_total // (16 * unroll)), _pack, None)

    def _zero(i, _):
        for u in range(unroll):
            slab[pl.ds(i * i32(16 * unroll) + i32(16 * u), 16)] = zeros16
        return _

    lax.fori_loop(i32(0), i32(slab_words // (16 * unroll)), _zero, None)

    base2 = wid * i32(2 * slab_words)
    one16 = jnp.full((16,), 1, jnp.int32)
    hi16 = jnp.full((16,), 1 << 16, jnp.int32)

    def _scan(e, _):
        for u in range(unroll):
            off = e * i32(16 * unroll) + i32(16 * u)
            local2 = flat_v[pl.ds(off, 16)] - base2
            inslab = plsc.bitcast(local2, jnp.uint32) < jnp.uint32(2 * slab_words)
            odd = (local2 & i32(1)) == i32(1)
            idx = jnp.where(inslab, local2 >> i32(1), i32(0))
            mlo = inslab & jnp.logical_not(odd)
            mhi = inslab & odd
            plsc.addupdate_scatter(slab, [idx], one16, mask=mlo)
            plsc.addupdate_scatter(slab, [idx], hi16, mask=mhi)
        return _

    lax.fori_loop(i32(0), i32(e_total // (16 * unroll)), _scan, None)

    pltpu.sync_copy(slab, a_hbm.at[pl.ds(wid * i32(slab_words), slab_words)])


def _sc_build_adj(dst_i32, src_i32):
    mesh = plsc.VectorSubcoreMesh(core_axis_name="c", subcore_axis_name="s")
    e_total = dst_i32.shape[0]
    f = functools.partial(
        pl.kernel,
        mesh=mesh,
        out_type=jax.ShapeDtypeStruct((N * (N // 2),), jnp.int32),
        scratch_types=[
            pltpu.VMEM((e_total,), jnp.int32),
            pltpu.VMEM((e_total,), jnp.int32),
            pltpu.VMEM(((N // 32) * (N // 2),), jnp.int32),
            pltpu.SemaphoreType.DMA,
        ],
        compiler_params=pltpu.CompilerParams(needs_layout_passes=False),
    )(_sc_adj_body)
    return f(dst_i32, src_i32)


def _fill_moments(x_ref, mo_ref):
    a = jnp.abs(x_ref[...])
    for b in range(a.shape[0]):
        p = a[b]
        for m in range(M):
            c0 = (b * M + m) * D
            mo_ref[:, c0:c0 + D] = p.astype(jnp.bfloat16)
            if m + 1 < M:
                p = p * a[b]


def _main_body(a_ref, x_ref, out_ref, abf, mo_ref):
    @pl.when(pl.program_id(0) == 0)
    def _cast():
        aw = a_ref[...]
        hw = N // 2
        abf[:, :hw] = ((aw & 0xFFFF) != 0).astype(jnp.bfloat16)
        abf[:, hw:] = ((aw >> 16) != 0).astype(jnp.bfloat16)
        _fill_moments(x_ref, mo_ref)

    rr = out_ref.shape[1]
    nb = out_ref.shape[0]
    row0 = pl.program_id(0) * rr
    a = abf[...]
    mo = mo_ref[...]
    hr = rr // 2
    halves = []
    for h in range(2):
        r0h = row0 + h * hr
        rows = lax.broadcasted_iota(jnp.int32, (hr, N), 0) + r0h
        cols = lax.broadcasted_iota(jnp.int32, (hr, N), 1)
        halves.append((rows == cols) | (abf[pl.ds(r0h, hr), :] > 0))
    for k in range(K):
        for h in range(2):
            r_bf = halves[h].astype(jnp.bfloat16)
            s = jnp.dot(r_bf, mo, preferred_element_type=jnp.float32)
            for b in range(nb):
                out_ref[b, h * hr:(h + 1) * hr, k] = (
                    s[:, b * M * D:(b + 1) * M * D].reshape(hr, M, D)
                )
            if k + 1 < K:
                ra = jnp.dot(r_bf, a, preferred_element_type=jnp.float32)
                halves[h] = halves[h] | (ra > 0.0)


def _tc_main(a_i32, x, nb):
    c = nb * M * D
    rr = 512
    return pl.pallas_call(
        _main_body,
        grid=(N // rr,),
        in_specs=[
            pl.BlockSpec((N, N // 2), lambda i: (i * 0, i * 0)),
            pl.BlockSpec((nb, N, D), lambda i: (i * 0, i * 0, i * 0)),
        ],
        out_specs=pl.BlockSpec(
            (nb, rr, K, M, D), lambda i: (i * 0, i, i * 0, i * 0, i * 0)
        ),
        out_shape=jax.ShapeDtypeStruct((nb, N, K, M, D), jnp.float32),
        scratch_shapes=[
            pltpu.VMEM((N, N), jnp.bfloat16),
            pltpu.VMEM((N, nb * M * D), jnp.bfloat16),
        ],
        compiler_params=pltpu.CompilerParams(
            vmem_limit_bytes=100 * 1024 * 1024,
        ),
    )(a_i32, x)


def kernel(x, edge_index):
    b = x.shape[0]
    e32 = edge_index.astype(jnp.int32)
    a_i32 = _sc_build_adj(e32[1], e32[0]).reshape(N, N // 2)
    return _tc_main(a_i32, x, b)
